# baseline (speedup 1.0000x reference)
import jax
import jax.numpy as jnp
from jax import lax
from jax.experimental import pallas as pl
from jax.experimental.pallas import tpu as pltpu
from jax.experimental.pallas import tpu_sc as plsc

_NC = 2
_NS = 16
_NW = _NC * _NS


def _sc_gather(ctab, idx_flat):
    total = idx_flat.shape[0]
    per_tile = total // _NW
    n_chunks = max(1, per_tile // 3072)
    while per_tile % n_chunks:
        n_chunks += 1
    chunk = per_tile // n_chunks

    mesh = plsc.VectorSubcoreMesh(
        core_axis_name="c", subcore_axis_name="s",
        num_cores=_NC, num_subcores=_NS)

    def body(ctab_hbm, idx_hbm, out_hbm, idx_v, rows_v, sem):
        wid = lax.axis_index("s") * _NC + lax.axis_index("c")
        base = wid * per_tile
        for c in range(n_chunks):
            off = base + c * chunk
            pltpu.sync_copy(idx_hbm.at[pl.ds(off, chunk)], idx_v)
            pltpu.async_copy(ctab_hbm.at[idx_v], rows_v, sem).wait()
            pltpu.sync_copy(rows_v, out_hbm.at[pl.ds(off, chunk)])

    f = pl.kernel(
        body,
        out_type=jax.ShapeDtypeStruct((total, 16), jnp.float32),
        mesh=mesh,
        scratch_types=[
            pltpu.VMEM((chunk,), jnp.int32),
            pltpu.VMEM((chunk, 16), jnp.float32),
            pltpu.SemaphoreType.DMA,
        ],
        compiler_params=pltpu.CompilerParams(use_tc_tiling_on_sc=False),
    )
    return f(ctab, idx_flat)


def _tc_body(g_ref, s12_ref, ehl_ref, plvl_ref, phl_ref, g17_ref, map_ref,
             pW1e_ref, pW1s_ref, pb1_ref, pW2_ref, pb2_ref,
             eW1e_ref, eW1h_ref, eb1_ref, eW2_ref, eb2_ref,
             paW_ref, pab_ref, gW_ref, gb_ref,
             fWp_ref, fWe_ref, fWpa_ref, fWm_ref, fWg_ref, fb_ref,
             out_ref):
    n = out_ref.shape[0]
    pW1e = pW1e_ref[...]
    pW1s = pW1s_ref[...]
    pb1 = pb1_ref[...]
    pW2 = pW2_ref[...]
    eW1e = eW1e_ref[...]
    eW1h = eW1h_ref[...]
    eb1 = eb1_ref[...]
    eW2 = eW2_ref[...]

    pacc = jnp.zeros((n, 128), jnp.float32)
    eacc = jnp.zeros((n, 128), jnp.float32)
    for p in range(6):
        blk = g_ref[p]
        emb = blk[:, 0:128]
        est = blk[:, 128:144]
        st = s12_ref[p]
        h1 = jax.nn.relu(
            jnp.dot(emb, pW1e, preferred_element_type=jnp.float32)
            + jnp.dot(st, pW1s, preferred_element_type=jnp.float32) + pb1)
        pacc = pacc + jnp.dot(h1, pW2, preferred_element_type=jnp.float32)

        eh = ehl_ref[p]
        g1 = jax.nn.relu(
            jnp.dot(est, eW1e, preferred_element_type=jnp.float32)
            + jnp.dot(eh, eW1h, preferred_element_type=jnp.float32) + eb1)
        eacc = eacc + jnp.dot(g1, eW2, preferred_element_type=jnp.float32)

    player = pacc * (1.0 / 6.0) + pb2_ref[...]
    enemy = eacc * (1.0 / 6.0) + eb2_ref[...]

    php = jnp.mean(phl_ref[...], axis=1, keepdims=True)
    plv = jnp.mean(plvl_ref[...].astype(jnp.float32), axis=1, keepdims=True)
    party = php * paW_ref[0:1, :] + plv * paW_ref[1:2, :] + pab_ref[...]

    gv = jax.nn.relu(jnp.dot(g17_ref[...], gW_ref[...],
                             preferred_element_type=jnp.float32)
                     + gb_ref[...])

    out = (jnp.dot(player, fWp_ref[...], preferred_element_type=jnp.float32)
           + jnp.dot(enemy, fWe_ref[...], preferred_element_type=jnp.float32)
           + jnp.dot(party, fWpa_ref[...], preferred_element_type=jnp.float32)
           + jnp.dot(map_ref[...], fWm_ref[...],
                     preferred_element_type=jnp.float32)
           + jnp.dot(gv, fWg_ref[...], preferred_element_type=jnp.float32)
           + fb_ref[...])
    out_ref[...] = jax.nn.relu(out)


def kernel(p_species, p_moves, p_ability, p_status, p_item, e_status,
           party_level, p_hp, p_lvl, p_att, p_defn, p_spe, p_spA, p_spD,
           p_pp, p_exp, e_hp, e_lvl, party_hp, inbattle, badge, hms,
           map_feat, species_emb, move_emb, ability_emb, status_emb,
           item_emb, e_status_emb, pW1, pb1, pW2, pb2, eW1, eb1, eW2, eb2,
           partyW, partyb, gW, gb, fW, fb):
    b = p_species.shape[0]
    f32 = jnp.float32

    ctab = jnp.concatenate([species_emb, move_emb, ability_emb, status_emb,
                            item_emb, e_status_emb], axis=0)
    idx = jnp.concatenate([
        p_species[..., None].astype(jnp.int32),
        p_moves.astype(jnp.int32) + 412,
        p_ability[..., None].astype(jnp.int32) + 767,
        p_status[..., None].astype(jnp.int32) + 845,
        p_item[..., None].astype(jnp.int32) + 851,
        e_status[..., None].astype(jnp.int32) + 1228,
    ], axis=-1)
    idx = idx.transpose(1, 0, 2).reshape(-1)

    gath = _sc_gather(ctab, idx)
    gath = gath.reshape(6, b, 144)

    s12 = jnp.stack([p_hp, p_lvl, p_att, p_defn, p_spe, p_spA, p_spD,
                     p_pp[:, :, 0], p_pp[:, :, 1], p_pp[:, :, 2],
                     p_pp[:, :, 3], p_exp], axis=-1)
    s12 = s12.transpose(1, 0, 2)
    ehl = jnp.stack([e_hp, e_lvl], axis=-1).transpose(1, 0, 2)
    g17 = jnp.concatenate([inbattle, badge, hms], axis=1)

    pW1e = pW1[0:128]
    pW1s = jnp.concatenate([
        pW1[128:129], pW1[129:130] / 100.0, pW1[130:135],
        jnp.repeat(pW1[135:136] / 4.0, 4, axis=0), pW1[136:137]], axis=0)
    eW1e = eW1[0:16]
    eW1h = jnp.concatenate([eW1[16:17], eW1[17:18] / 100.0], axis=0)
    paW = jnp.concatenate([partyW[0:1], partyW[1:2] / 100.0], axis=0)
    fWp = fW[0:128]
    fWe = fW[128:256]
    fWpa = fW[256:384]
    fWm = fW[384:439]
    fWg = fW[439:471]

    n = min(512, b)
    grid = (b // n,)

    def bspec(k):
        return pl.BlockSpec((n, k), lambda i: (i, 0))

    def pspec(k):
        return pl.BlockSpec((6, n, k), lambda i: (0, i, 0))

    def wspec(shape):
        nd = len(shape)
        return pl.BlockSpec(shape, lambda i: (0,) * nd)

    weights = [pW1e, pW1s, pb1.reshape(1, 128), pW2, pb2.reshape(1, 128),
               eW1e, eW1h, eb1.reshape(1, 128), eW2, eb2.reshape(1, 128),
               paW, partyb.reshape(1, 128), gW, gb.reshape(1, 32),
               fWp, fWe, fWpa, fWm, fWg, fb.reshape(1, 256)]

    in_specs = ([pspec(144), pspec(12), pspec(2), bspec(6), bspec(6),
                 bspec(17), bspec(55)]
                + [wspec(w.shape) for w in weights])

    return pl.pallas_call(
        _tc_body,
        grid=grid,
        in_specs=in_specs,
        out_specs=pl.BlockSpec((n, 256), lambda i: (i, 0)),
        out_shape=jax.ShapeDtypeStruct((b, 256), f32),
        compiler_params=pltpu.CompilerParams(
            dimension_semantics=("parallel",)),
    )(gath, s12, ehl, party_level.astype(jnp.int32), party_hp, g17,
      map_feat, *weights)


# speedup vs baseline: 1.2175x; 1.2175x over previous
import jax
import jax.numpy as jnp
from jax import lax
from jax.experimental import pallas as pl
from jax.experimental.pallas import tpu as pltpu
from jax.experimental.pallas import tpu_sc as plsc

_NC = 2
_NS = 16
_NW = _NC * _NS


def _sc_gather(ctab, idx_flat):
    total = idx_flat.shape[0]
    per_tile = total // _NW
    n_chunks = max(1, per_tile // 3072)
    while per_tile % n_chunks:
        n_chunks += 1
    chunk = per_tile // n_chunks

    mesh = plsc.VectorSubcoreMesh(
        core_axis_name="c", subcore_axis_name="s",
        num_cores=_NC, num_subcores=_NS)
    v = ctab.shape[0]

    def body(ctab_hbm, idx_hbm, out_hbm, tab_sh, idx_v, rows_v, gsem, osem):
        sid = lax.axis_index("s")
        wid = sid * _NC + lax.axis_index("c")
        base = wid * per_tile

        @pl.when(sid == 0)
        def _():
            pltpu.sync_copy(ctab_hbm, tab_sh)
        plsc.subcore_barrier()

        def wb_slice(c):
            return out_hbm.at[pl.ds(base + c * chunk, chunk)]

        pltpu.sync_copy(idx_hbm.at[pl.ds(base, chunk)], idx_v.at[0])
        pltpu.async_copy(tab_sh.at[idx_v.at[0]], rows_v.at[0], gsem)
        for c in range(n_chunks):
            cur = c % 2
            nxt = 1 - cur
            if c + 1 < n_chunks:
                pltpu.sync_copy(idx_hbm.at[pl.ds(base + (c + 1) * chunk,
                                                 chunk)],
                                idx_v.at[nxt])
                if c >= 1:
                    pltpu.make_async_copy(rows_v.at[nxt], wb_slice(c - 1),
                                          osem).wait()
                pltpu.async_copy(tab_sh.at[idx_v.at[nxt]], rows_v.at[nxt],
                                 gsem)
            pltpu.make_async_copy(tab_sh.at[idx_v.at[cur]], rows_v.at[cur],
                                  gsem).wait()
            pltpu.async_copy(rows_v.at[cur], wb_slice(c), osem)
        pltpu.make_async_copy(rows_v.at[(n_chunks - 1) % 2],
                              wb_slice(n_chunks - 1), osem).wait()
        if n_chunks >= 2:
            pltpu.make_async_copy(rows_v.at[n_chunks % 2],
                                  wb_slice(n_chunks - 2), osem).wait()

    f = pl.kernel(
        body,
        out_type=jax.ShapeDtypeStruct((total, 16), jnp.float32),
        mesh=mesh,
        scratch_types=[
            pltpu.VMEM_SHARED((v, 16), jnp.float32),
            pltpu.VMEM((2, chunk), jnp.int32),
            pltpu.VMEM((2, chunk, 16), jnp.float32),
            pltpu.SemaphoreType.DMA,
            pltpu.SemaphoreType.DMA,
        ],
        compiler_params=pltpu.CompilerParams(use_tc_tiling_on_sc=False),
    )
    return f(ctab, idx_flat)


def _tc_body(g_ref, s12_ref, ehl_ref, plvl_ref, phl_ref, g17_ref, map_ref,
             pW1e_ref, pW1s_ref, pb1_ref, pW2_ref, pb2_ref,
             eW1e_ref, eW1h_ref, eb1_ref, eW2_ref, eb2_ref,
             paW_ref, pab_ref, gW_ref, gb_ref,
             fWp_ref, fWe_ref, fWpa_ref, fWm_ref, fWg_ref, fb_ref,
             out_ref):
    n = out_ref.shape[0]
    pW1e = pW1e_ref[...]
    pW1s = pW1s_ref[...]
    pb1 = pb1_ref[...]
    pW2 = pW2_ref[...]
    eW1e = eW1e_ref[...]
    eW1h = eW1h_ref[...]
    eb1 = eb1_ref[...]
    eW2 = eW2_ref[...]

    pacc = jnp.zeros((n, 128), jnp.float32)
    eacc = jnp.zeros((n, 128), jnp.float32)
    for p in range(6):
        blk = g_ref[p]
        emb = blk[:, 0:128]
        est = blk[:, 128:144]
        st = s12_ref[p]
        h1 = jax.nn.relu(
            jnp.dot(emb, pW1e, preferred_element_type=jnp.float32)
            + jnp.dot(st, pW1s, preferred_element_type=jnp.float32) + pb1)
        pacc = pacc + jnp.dot(h1, pW2, preferred_element_type=jnp.float32)

        eh = ehl_ref[p]
        g1 = jax.nn.relu(
            jnp.dot(est, eW1e, preferred_element_type=jnp.float32)
            + jnp.dot(eh, eW1h, preferred_element_type=jnp.float32) + eb1)
        eacc = eacc + jnp.dot(g1, eW2, preferred_element_type=jnp.float32)

    player = pacc * (1.0 / 6.0) + pb2_ref[...]
    enemy = eacc * (1.0 / 6.0) + eb2_ref[...]

    php = jnp.mean(phl_ref[...], axis=1, keepdims=True)
    plv = jnp.mean(plvl_ref[...].astype(jnp.float32), axis=1, keepdims=True)
    party = php * paW_ref[0:1, :] + plv * paW_ref[1:2, :] + pab_ref[...]

    gv = jax.nn.relu(jnp.dot(g17_ref[...], gW_ref[...],
                             preferred_element_type=jnp.float32)
                     + gb_ref[...])

    out = (jnp.dot(player, fWp_ref[...], preferred_element_type=jnp.float32)
           + jnp.dot(enemy, fWe_ref[...], preferred_element_type=jnp.float32)
           + jnp.dot(party, fWpa_ref[...], preferred_element_type=jnp.float32)
           + jnp.dot(map_ref[...], fWm_ref[...],
                     preferred_element_type=jnp.float32)
           + jnp.dot(gv, fWg_ref[...], preferred_element_type=jnp.float32)
           + fb_ref[...])
    out_ref[...] = jax.nn.relu(out)


def kernel(p_species, p_moves, p_ability, p_status, p_item, e_status,
           party_level, p_hp, p_lvl, p_att, p_defn, p_spe, p_spA, p_spD,
           p_pp, p_exp, e_hp, e_lvl, party_hp, inbattle, badge, hms,
           map_feat, species_emb, move_emb, ability_emb, status_emb,
           item_emb, e_status_emb, pW1, pb1, pW2, pb2, eW1, eb1, eW2, eb2,
           partyW, partyb, gW, gb, fW, fb):
    b = p_species.shape[0]
    f32 = jnp.float32

    ctab = jnp.concatenate([species_emb, move_emb, ability_emb, status_emb,
                            item_emb, e_status_emb], axis=0)
    idx = jnp.concatenate([
        p_species[..., None].astype(jnp.int32),
        p_moves.astype(jnp.int32) + 412,
        p_ability[..., None].astype(jnp.int32) + 767,
        p_status[..., None].astype(jnp.int32) + 845,
        p_item[..., None].astype(jnp.int32) + 851,
        e_status[..., None].astype(jnp.int32) + 1228,
    ], axis=-1)
    idx = idx.transpose(1, 0, 2).reshape(-1)

    gath = _sc_gather(ctab, idx)
    gath = gath.reshape(6, b, 144)

    s12 = jnp.stack([p_hp, p_lvl, p_att, p_defn, p_spe, p_spA, p_spD,
                     p_pp[:, :, 0], p_pp[:, :, 1], p_pp[:, :, 2],
                     p_pp[:, :, 3], p_exp], axis=-1)
    s12 = s12.transpose(1, 0, 2)
    ehl = jnp.stack([e_hp, e_lvl], axis=-1).transpose(1, 0, 2)
    g17 = jnp.concatenate([inbattle, badge, hms], axis=1)

    pW1e = pW1[0:128]
    pW1s = jnp.concatenate([
        pW1[128:129], pW1[129:130] / 100.0, pW1[130:135],
        jnp.repeat(pW1[135:136] / 4.0, 4, axis=0), pW1[136:137]], axis=0)
    eW1e = eW1[0:16]
    eW1h = jnp.concatenate([eW1[16:17], eW1[17:18] / 100.0], axis=0)
    paW = jnp.concatenate([partyW[0:1], partyW[1:2] / 100.0], axis=0)
    fWp = fW[0:128]
    fWe = fW[128:256]
    fWpa = fW[256:384]
    fWm = fW[384:439]
    fWg = fW[439:471]

    n = min(512, b)
    grid = (b // n,)

    def bspec(k):
        return pl.BlockSpec((n, k), lambda i: (i, 0))

    def pspec(k):
        return pl.BlockSpec((6, n, k), lambda i: (0, i, 0))

    def wspec(shape):
        nd = len(shape)
        return pl.BlockSpec(shape, lambda i: (0,) * nd)

    weights = [pW1e, pW1s, pb1.reshape(1, 128), pW2, pb2.reshape(1, 128),
               eW1e, eW1h, eb1.reshape(1, 128), eW2, eb2.reshape(1, 128),
               paW, partyb.reshape(1, 128), gW, gb.reshape(1, 32),
               fWp, fWe, fWpa, fWm, fWg, fb.reshape(1, 256)]

    in_specs = ([pspec(144), pspec(12), pspec(2), bspec(6), bspec(6),
                 bspec(17), bspec(55)]
                + [wspec(w.shape) for w in weights])

    return pl.pallas_call(
        _tc_body,
        grid=grid,
        in_specs=in_specs,
        out_specs=pl.BlockSpec((n, 256), lambda i: (i, 0)),
        out_shape=jax.ShapeDtypeStruct((b, 256), f32),
        compiler_params=pltpu.CompilerParams(
            dimension_semantics=("parallel",)),
    )(gath, s12, ehl, party_level.astype(jnp.int32), party_hp, g17,
      map_feat, *weights)


# speedup vs baseline: 1.2180x; 1.0004x over previous
import jax
import jax.numpy as jnp
from jax import lax
from jax.experimental import pallas as pl
from jax.experimental.pallas import tpu as pltpu
from jax.experimental.pallas import tpu_sc as plsc

_NC = 2
_NS = 16
_NW = _NC * _NS


def _sc_gather(ctab, idx_flat):
    total = idx_flat.shape[0]
    per_tile = total // _NW
    n_chunks = max(1, -(-per_tile // 1536))
    while per_tile % n_chunks:
        n_chunks += 1
    chunk = per_tile // n_chunks

    mesh = plsc.VectorSubcoreMesh(
        core_axis_name="c", subcore_axis_name="s",
        num_cores=_NC, num_subcores=_NS)
    v = ctab.shape[0]

    n_groups = chunk // 16
    assert chunk % 16 == 0

    def body(ctab_hbm, idx_hbm, out_hbm, tab_v, idx_v, rows_v, osem):
        wid = lax.axis_index("s") * _NC + lax.axis_index("c")
        base = wid * per_tile

        pltpu.sync_copy(ctab_hbm, tab_v)
        iota16 = lax.iota(jnp.int32, 16)

        def wb_slice(c):
            return out_hbm.at[pl.ds((base + c * chunk) * 16, chunk * 16)]

        for c in range(n_chunks):
            cur = c % 2
            if c >= 2:
                pltpu.make_async_copy(rows_v.at[cur], wb_slice(c - 2),
                                      osem).wait()
            pltpu.sync_copy(idx_hbm.at[pl.ds(base + c * chunk, chunk)],
                            idx_v)
            rv = rows_v.at[cur]

            def bg(g, carry):
                addr = idx_v[pl.ds(g * 16, 16)] * 16
                obase = g * 256 + iota16 * 16
                for d in range(16):
                    vals = plsc.load_gather(tab_v, [addr + d])
                    plsc.store_scatter(rv, [obase + d], vals)
                return carry

            lax.fori_loop(0, n_groups, bg, 0, unroll=2)
            pltpu.async_copy(rv, wb_slice(c), osem)
        pltpu.make_async_copy(rows_v.at[(n_chunks - 1) % 2],
                              wb_slice(n_chunks - 1), osem).wait()
        if n_chunks >= 2:
            pltpu.make_async_copy(rows_v.at[n_chunks % 2],
                                  wb_slice(n_chunks - 2), osem).wait()

    f = pl.kernel(
        body,
        out_type=jax.ShapeDtypeStruct((total * 16,), jnp.float32),
        mesh=mesh,
        scratch_types=[
            pltpu.VMEM((v * 16,), jnp.float32),
            pltpu.VMEM((chunk,), jnp.int32),
            pltpu.VMEM((2, chunk * 16), jnp.float32),
            pltpu.SemaphoreType.DMA,
        ],
        compiler_params=pltpu.CompilerParams(use_tc_tiling_on_sc=False,
                                             needs_layout_passes=False),
    )
    return f(ctab.reshape(-1), idx_flat).reshape(total, 16)


def _tc_body(g_ref, s12_ref, ehl_ref, plvl_ref, phl_ref, g17_ref, map_ref,
             pW1e_ref, pW1s_ref, pb1_ref, pW2_ref, pb2_ref,
             eW1e_ref, eW1h_ref, eb1_ref, eW2_ref, eb2_ref,
             paW_ref, pab_ref, gW_ref, gb_ref,
             fWp_ref, fWe_ref, fWpa_ref, fWm_ref, fWg_ref, fb_ref,
             out_ref):
    n = out_ref.shape[0]
    pW1e = pW1e_ref[...]
    pW1s = pW1s_ref[...]
    pb1 = pb1_ref[...]
    pW2 = pW2_ref[...]
    eW1e = eW1e_ref[...]
    eW1h = eW1h_ref[...]
    eb1 = eb1_ref[...]
    eW2 = eW2_ref[...]

    pacc = jnp.zeros((n, 128), jnp.float32)
    eacc = jnp.zeros((n, 128), jnp.float32)
    for p in range(6):
        blk = g_ref[p]
        emb = blk[:, 0:128]
        est = blk[:, 128:144]
        st = s12_ref[p]
        h1 = jax.nn.relu(
            jnp.dot(emb, pW1e, preferred_element_type=jnp.float32)
            + jnp.dot(st, pW1s, preferred_element_type=jnp.float32) + pb1)
        pacc = pacc + jnp.dot(h1, pW2, preferred_element_type=jnp.float32)

        eh = ehl_ref[p]
        g1 = jax.nn.relu(
            jnp.dot(est, eW1e, preferred_element_type=jnp.float32)
            + jnp.dot(eh, eW1h, preferred_element_type=jnp.float32) + eb1)
        eacc = eacc + jnp.dot(g1, eW2, preferred_element_type=jnp.float32)

    player = pacc * (1.0 / 6.0) + pb2_ref[...]
    enemy = eacc * (1.0 / 6.0) + eb2_ref[...]

    php = jnp.mean(phl_ref[...], axis=1, keepdims=True)
    plv = jnp.mean(plvl_ref[...].astype(jnp.float32), axis=1, keepdims=True)
    party = php * paW_ref[0:1, :] + plv * paW_ref[1:2, :] + pab_ref[...]

    gv = jax.nn.relu(jnp.dot(g17_ref[...], gW_ref[...],
                             preferred_element_type=jnp.float32)
                     + gb_ref[...])

    out = (jnp.dot(player, fWp_ref[...], preferred_element_type=jnp.float32)
           + jnp.dot(enemy, fWe_ref[...], preferred_element_type=jnp.float32)
           + jnp.dot(party, fWpa_ref[...], preferred_element_type=jnp.float32)
           + jnp.dot(map_ref[...], fWm_ref[...],
                     preferred_element_type=jnp.float32)
           + jnp.dot(gv, fWg_ref[...], preferred_element_type=jnp.float32)
           + fb_ref[...])
    out_ref[...] = jax.nn.relu(out)


def kernel(p_species, p_moves, p_ability, p_status, p_item, e_status,
           party_level, p_hp, p_lvl, p_att, p_defn, p_spe, p_spA, p_spD,
           p_pp, p_exp, e_hp, e_lvl, party_hp, inbattle, badge, hms,
           map_feat, species_emb, move_emb, ability_emb, status_emb,
           item_emb, e_status_emb, pW1, pb1, pW2, pb2, eW1, eb1, eW2, eb2,
           partyW, partyb, gW, gb, fW, fb):
    b = p_species.shape[0]
    f32 = jnp.float32

    ctab = jnp.concatenate([species_emb, move_emb, ability_emb, status_emb,
                            item_emb, e_status_emb], axis=0)
    idx = jnp.concatenate([
        p_species[..., None].astype(jnp.int32),
        p_moves.astype(jnp.int32) + 412,
        p_ability[..., None].astype(jnp.int32) + 767,
        p_status[..., None].astype(jnp.int32) + 845,
        p_item[..., None].astype(jnp.int32) + 851,
        e_status[..., None].astype(jnp.int32) + 1228,
    ], axis=-1)
    idx = idx.transpose(1, 0, 2).reshape(-1)

    gath = _sc_gather(ctab, idx)
    gath = gath.reshape(6, b, 144)

    s12 = jnp.stack([p_hp, p_lvl, p_att, p_defn, p_spe, p_spA, p_spD,
                     p_pp[:, :, 0], p_pp[:, :, 1], p_pp[:, :, 2],
                     p_pp[:, :, 3], p_exp], axis=-1)
    s12 = s12.transpose(1, 0, 2)
    ehl = jnp.stack([e_hp, e_lvl], axis=-1).transpose(1, 0, 2)
    g17 = jnp.concatenate([inbattle, badge, hms], axis=1)

    pW1e = pW1[0:128]
    pW1s = jnp.concatenate([
        pW1[128:129], pW1[129:130] / 100.0, pW1[130:135],
        jnp.repeat(pW1[135:136] / 4.0, 4, axis=0), pW1[136:137]], axis=0)
    eW1e = eW1[0:16]
    eW1h = jnp.concatenate([eW1[16:17], eW1[17:18] / 100.0], axis=0)
    paW = jnp.concatenate([partyW[0:1], partyW[1:2] / 100.0], axis=0)
    fWp = fW[0:128]
    fWe = fW[128:256]
    fWpa = fW[256:384]
    fWm = fW[384:439]
    fWg = fW[439:471]

    n = min(512, b)
    grid = (b // n,)

    def bspec(k):
        return pl.BlockSpec((n, k), lambda i: (i, 0))

    def pspec(k):
        return pl.BlockSpec((6, n, k), lambda i: (0, i, 0))

    def wspec(shape):
        nd = len(shape)
        return pl.BlockSpec(shape, lambda i: (0,) * nd)

    weights = [pW1e, pW1s, pb1.reshape(1, 128), pW2, pb2.reshape(1, 128),
               eW1e, eW1h, eb1.reshape(1, 128), eW2, eb2.reshape(1, 128),
               paW, partyb.reshape(1, 128), gW, gb.reshape(1, 32),
               fWp, fWe, fWpa, fWm, fWg, fb.reshape(1, 256)]

    in_specs = ([pspec(144), pspec(12), pspec(2), bspec(6), bspec(6),
                 bspec(17), bspec(55)]
                + [wspec(w.shape) for w in weights])

    return pl.pallas_call(
        _tc_body,
        grid=grid,
        in_specs=in_specs,
        out_specs=pl.BlockSpec((n, 256), lambda i: (i, 0)),
        out_shape=jax.ShapeDtypeStruct((b, 256), f32),
        compiler_params=pltpu.CompilerParams(
            dimension_semantics=("parallel",)),
    )(gath, s12, ehl, party_level.astype(jnp.int32), party_hp, g17,
      map_feat, *weights)


# speedup vs baseline: 1.7861x; 1.4665x over previous
import jax
import jax.numpy as jnp
from jax import lax
from jax.experimental import pallas as pl
from jax.experimental.pallas import tpu as pltpu
from jax.experimental.pallas import tpu_sc as plsc

_NC = 2
_NS = 16
_NW = _NC * _NS


def _sc_gather(ctab, idx_px, idx_est):
    v = ctab.shape[0]
    lp = idx_px.shape[0]
    le = idx_est.shape[0]

    def plan(total):
        per_tile = total // _NW
        n_chunks = max(1, -(-per_tile // 1536))
        while per_tile % n_chunks:
            n_chunks += 1
        return per_tile, n_chunks, per_tile // n_chunks

    pp, pnc, pch = plan(lp)
    ep, enc, ech = plan(le)
    chunk = max(pch, ech)

    mesh = plsc.VectorSubcoreMesh(
        core_axis_name="c", subcore_axis_name="s",
        num_cores=_NC, num_subcores=_NS)

    def body(ctab_hbm, ipx_hbm, iest_hbm, opx_hbm, oest_hbm,
             tab_v, idx_v, rows_v, osem):
        wid = lax.axis_index("s") * _NC + lax.axis_index("c")
        pltpu.sync_copy(ctab_hbm, tab_v)
        iota16 = lax.iota(jnp.int32, 16)

        def run(idx_hbm, out_hbm, per_tile, n_chunks, ch, buf0):
            base = wid * per_tile

            def wb_slice(c):
                return out_hbm.at[pl.ds((base + c * ch) * 16, ch * 16)]

            for c in range(n_chunks):
                cur = (buf0 + c) % 2
                if c >= 2:
                    pltpu.make_async_copy(
                        rows_v.at[cur, pl.ds(0, ch * 16)], wb_slice(c - 2),
                        osem).wait()
                pltpu.sync_copy(idx_hbm.at[pl.ds(base + c * ch, ch)],
                                idx_v.at[pl.ds(0, ch)])
                rv = rows_v.at[cur, pl.ds(0, ch * 16)]

                def bg(g, carry):
                    addr = idx_v[pl.ds(g * 16, 16)] * 16
                    obase = g * 256 + iota16 * 16
                    for d in range(16):
                        vals = plsc.load_gather(tab_v, [addr + d])
                        plsc.store_scatter(rv, [obase + d], vals)
                    return carry

                lax.fori_loop(0, ch // 16, bg, 0, unroll=2)
                pltpu.async_copy(rv, wb_slice(c), osem)
            pltpu.make_async_copy(
                rows_v.at[(buf0 + n_chunks - 1) % 2, pl.ds(0, ch * 16)],
                wb_slice(n_chunks - 1), osem).wait()
            if n_chunks >= 2:
                pltpu.make_async_copy(
                    rows_v.at[(buf0 + n_chunks) % 2, pl.ds(0, ch * 16)],
                    wb_slice(n_chunks - 2), osem).wait()
            return (buf0 + n_chunks) % 2

        nb = run(ipx_hbm, opx_hbm, pp, pnc, pch, 0)
        run(iest_hbm, oest_hbm, ep, enc, ech, nb)

    f = pl.kernel(
        body,
        out_type=(jax.ShapeDtypeStruct((lp * 16,), jnp.float32),
                  jax.ShapeDtypeStruct((le * 16,), jnp.float32)),
        mesh=mesh,
        scratch_types=[
            pltpu.VMEM((v * 16,), jnp.float32),
            pltpu.VMEM((chunk,), jnp.int32),
            pltpu.VMEM((2, chunk * 16), jnp.float32),
            pltpu.SemaphoreType.DMA,
        ],
        compiler_params=pltpu.CompilerParams(use_tc_tiling_on_sc=False,
                                             needs_layout_passes=False),
    )
    return f(ctab.reshape(-1), idx_px, idx_est)


def _tc_body(g_ref, e_ref, hp_ref, lvl_ref, att_ref, defn_ref, spe_ref,
             spA_ref, spD_ref, pp_ref, exp_ref, ehp_ref, elvl_ref,
             plvl_ref, phl_ref, inb_ref, badge_ref, hms_ref, map_ref,
             pW1e_ref, pW1s_ref, pb1_ref, pW2_ref, pb2_ref,
             eW1e_ref, eW1h_ref, eb1_ref, eW2_ref, eb2_ref,
             paW_ref, pab_ref, gW_ref, gb_ref,
             fWp_ref, fWe_ref, fWpa_ref, fWm_ref, fWg_ref, fb_ref,
             out_ref):
    n = out_ref.shape[0]
    pW1e = pW1e_ref[...]
    pW1s = pW1s_ref[...]
    pb1 = pb1_ref[...]
    pW2 = pW2_ref[...]
    eW1e = eW1e_ref[...]
    eW1h = eW1h_ref[...]
    eb1 = eb1_ref[...]
    eW2 = eW2_ref[...]

    pacc = jnp.zeros((n, 128), jnp.float32)
    eacc = jnp.zeros((n, 128), jnp.float32)
    for p in range(6):
        emb = g_ref[p]
        est = e_ref[p]
        pp_mean = jnp.mean(pp_ref[:, 4 * p:4 * p + 4], axis=1,
                           keepdims=True)
        stats = jnp.concatenate([
            hp_ref[:, p:p + 1], lvl_ref[:, p:p + 1] / 100.0,
            att_ref[:, p:p + 1], defn_ref[:, p:p + 1], spe_ref[:, p:p + 1],
            spA_ref[:, p:p + 1], spD_ref[:, p:p + 1], pp_mean,
            exp_ref[:, p:p + 1]], axis=1)
        h1 = jax.nn.relu(
            jnp.dot(emb, pW1e, preferred_element_type=jnp.float32)
            + jnp.dot(stats, pW1s, preferred_element_type=jnp.float32)
            + pb1)
        pacc = pacc + jnp.dot(h1, pW2, preferred_element_type=jnp.float32)

        eh = jnp.concatenate([ehp_ref[:, p:p + 1],
                              elvl_ref[:, p:p + 1] / 100.0], axis=1)
        g1 = jax.nn.relu(
            jnp.dot(est, eW1e, preferred_element_type=jnp.float32)
            + jnp.dot(eh, eW1h, preferred_element_type=jnp.float32) + eb1)
        eacc = eacc + jnp.dot(g1, eW2, preferred_element_type=jnp.float32)

    player = pacc * (1.0 / 6.0) + pb2_ref[...]
    enemy = eacc * (1.0 / 6.0) + eb2_ref[...]

    php = jnp.mean(phl_ref[...], axis=1, keepdims=True)
    plv = jnp.mean(plvl_ref[...].astype(jnp.float32) / 100.0, axis=1,
                   keepdims=True)
    party = php * paW_ref[0:1, :] + plv * paW_ref[1:2, :] + pab_ref[...]

    g_in = jnp.concatenate([inb_ref[...], badge_ref[...], hms_ref[...]],
                           axis=1)
    gv = jax.nn.relu(jnp.dot(g_in, gW_ref[...],
                             preferred_element_type=jnp.float32)
                     + gb_ref[...])

    out = (jnp.dot(player, fWp_ref[...], preferred_element_type=jnp.float32)
           + jnp.dot(enemy, fWe_ref[...], preferred_element_type=jnp.float32)
           + jnp.dot(party, fWpa_ref[...], preferred_element_type=jnp.float32)
           + jnp.dot(map_ref[...], fWm_ref[...],
                     preferred_element_type=jnp.float32)
           + jnp.dot(gv, fWg_ref[...], preferred_element_type=jnp.float32)
           + fb_ref[...])
    out_ref[...] = jax.nn.relu(out)


def kernel(p_species, p_moves, p_ability, p_status, p_item, e_status,
           party_level, p_hp, p_lvl, p_att, p_defn, p_spe, p_spA, p_spD,
           p_pp, p_exp, e_hp, e_lvl, party_hp, inbattle, badge, hms,
           map_feat, species_emb, move_emb, ability_emb, status_emb,
           item_emb, e_status_emb, pW1, pb1, pW2, pb2, eW1, eb1, eW2, eb2,
           partyW, partyb, gW, gb, fW, fb):
    b = p_species.shape[0]
    f32 = jnp.float32

    ctab = jnp.concatenate([species_emb, move_emb, ability_emb, status_emb,
                            item_emb, e_status_emb], axis=0)
    idx_px = jnp.concatenate([
        p_species[..., None].astype(jnp.int32),
        p_moves.astype(jnp.int32) + 412,
        p_ability[..., None].astype(jnp.int32) + 767,
        p_status[..., None].astype(jnp.int32) + 845,
        p_item[..., None].astype(jnp.int32) + 851,
    ], axis=-1)
    idx_px = idx_px.transpose(1, 0, 2).reshape(-1)
    idx_est = (e_status.astype(jnp.int32) + 1228).T.reshape(-1)

    gpx, gest = _sc_gather(ctab, idx_px, idx_est)
    gpx = gpx.reshape(6, b, 128)
    gest = gest.reshape(6, b, 16)

    pp = p_pp.reshape(b, 24)

    n = min(512, b)
    grid = (b // n,)

    def bspec(k):
        return pl.BlockSpec((n, k), lambda i: (i, 0))

    def pspec(k):
        return pl.BlockSpec((6, n, k), lambda i: (0, i, 0))

    def wspec(shape):
        nd = len(shape)
        return pl.BlockSpec(shape, lambda i: (0,) * nd)

    weights = [pW1[0:128], pW1[128:137], pb1.reshape(1, 128), pW2,
               pb2.reshape(1, 128), eW1[0:16], eW1[16:18],
               eb1.reshape(1, 128), eW2, eb2.reshape(1, 128),
               partyW, partyb.reshape(1, 128), gW, gb.reshape(1, 32),
               fW[0:128], fW[128:256], fW[256:384], fW[384:439],
               fW[439:471], fb.reshape(1, 256)]

    batch_in = [(p_hp, 6), (p_lvl, 6), (p_att, 6), (p_defn, 6), (p_spe, 6),
                (p_spA, 6), (p_spD, 6), (pp, 24), (p_exp, 6), (e_hp, 6),
                (e_lvl, 6), (party_level.astype(jnp.int32), 6),
                (party_hp, 6), (inbattle, 1), (badge, 8), (hms, 8),
                (map_feat, 55)]

    in_specs = ([pspec(128), pspec(16)]
                + [bspec(k) for _, k in batch_in]
                + [wspec(w.shape) for w in weights])

    return pl.pallas_call(
        _tc_body,
        grid=grid,
        in_specs=in_specs,
        out_specs=pl.BlockSpec((n, 256), lambda i: (i, 0)),
        out_shape=jax.ShapeDtypeStruct((b, 256), f32),
        compiler_params=pltpu.CompilerParams(
            dimension_semantics=("parallel",)),
    )(gpx, gest, *[a for a, _ in batch_in], *weights)


# speedup vs baseline: 2.8351x; 1.5873x over previous
import jax
import jax.numpy as jnp
from jax import lax
from jax.experimental import pallas as pl
from jax.experimental.pallas import tpu as pltpu
from jax.experimental.pallas import tpu_sc as plsc

_NC = 2
_NS = 16
_NW = _NC * _NS


def _sc_gather(ctab, idx_px, idx_est):
    v = ctab.shape[0]
    lp = idx_px.shape[0]
    le = idx_est.shape[0]

    def plan(total):
        per_tile = total // _NW
        n_chunks = max(1, -(-per_tile // 1536))
        while per_tile % n_chunks:
            n_chunks += 1
        return per_tile, n_chunks, per_tile // n_chunks

    pp, pnc, pch = plan(lp)
    ep, enc, ech = plan(le)
    chunk = max(pch, ech)

    mesh = plsc.VectorSubcoreMesh(
        core_axis_name="c", subcore_axis_name="s",
        num_cores=_NC, num_subcores=_NS)

    def body(ctab_hbm, ipx_hbm, iest_hbm, opx_hbm, oest_hbm,
             tab_v, idx_v, rows_v, osem):
        wid = lax.axis_index("s") * _NC + lax.axis_index("c")
        pltpu.sync_copy(ctab_hbm, tab_v)
        iota16 = lax.iota(jnp.int32, 16)
        rots = [jnp.bitwise_and(iota16 + d, 15) for d in range(16)]

        def run(idx_hbm, out_hbm, per_tile, n_chunks, ch, buf0):
            base = wid * per_tile

            def wb_slice(c):
                return out_hbm.at[pl.ds((base + c * ch) * 16, ch * 16)]

            for c in range(n_chunks):
                cur = (buf0 + c) % 2
                if c >= 2:
                    pltpu.make_async_copy(
                        rows_v.at[cur, pl.ds(0, ch * 16)], wb_slice(c - 2),
                        osem).wait()
                pltpu.sync_copy(idx_hbm.at[pl.ds(base + c * ch, ch)],
                                idx_v.at[pl.ds(0, ch)])
                rv = rows_v.at[cur, pl.ds(0, ch * 16)]

                def bg(g, carry):
                    addr = idx_v[pl.ds(g * 16, 16)] * 16
                    obase = g * 256 + iota16 * 16
                    for d in range(16):
                        rot = rots[d]
                        vals = plsc.load_gather(tab_v, [addr + rot])
                        plsc.store_scatter(rv, [obase + rot], vals)
                    return carry

                lax.fori_loop(0, ch // 16, bg, 0, unroll=2)
                pltpu.async_copy(rv, wb_slice(c), osem)
            pltpu.make_async_copy(
                rows_v.at[(buf0 + n_chunks - 1) % 2, pl.ds(0, ch * 16)],
                wb_slice(n_chunks - 1), osem).wait()
            if n_chunks >= 2:
                pltpu.make_async_copy(
                    rows_v.at[(buf0 + n_chunks) % 2, pl.ds(0, ch * 16)],
                    wb_slice(n_chunks - 2), osem).wait()
            return (buf0 + n_chunks) % 2

        nb = run(ipx_hbm, opx_hbm, pp, pnc, pch, 0)
        run(iest_hbm, oest_hbm, ep, enc, ech, nb)

    f = pl.kernel(
        body,
        out_type=(jax.ShapeDtypeStruct((lp * 16,), jnp.float32),
                  jax.ShapeDtypeStruct((le * 16,), jnp.float32)),
        mesh=mesh,
        scratch_types=[
            pltpu.VMEM((v * 16,), jnp.float32),
            pltpu.VMEM((chunk,), jnp.int32),
            pltpu.VMEM((2, chunk * 16), jnp.float32),
            pltpu.SemaphoreType.DMA,
        ],
        compiler_params=pltpu.CompilerParams(use_tc_tiling_on_sc=False,
                                             needs_layout_passes=False),
    )
    return f(ctab.reshape(-1), idx_px, idx_est)


def _tc_body(g_ref, e_ref, r_ref, plvl_ref, phl_ref, inb_ref, badge_ref,
             hms_ref, map_ref,
             pW1e_ref, G2_ref, pb1_ref, pW2_ref, pb2_ref,
             eW1e_ref, eb1_ref, eW2_ref, eb2_ref,
             paW_ref, pab_ref, gW_ref, gb_ref,
             fWp_ref, fWe_ref, fWpa_ref, fWm_ref, fWg_ref, fb_ref,
             out_ref):
    n = out_ref.shape[0]
    pW1e = pW1e_ref[...]
    pb1 = pb1_ref[...]
    pW2 = pW2_ref[...]
    eW1e = eW1e_ref[...]
    eb1 = eb1_ref[...]
    eW2 = eW2_ref[...]
    r = r_ref[...]

    pacc = jnp.zeros((n, 128), jnp.float32)
    eacc = jnp.zeros((n, 128), jnp.float32)
    for p in range(6):
        emb = g_ref[p]
        est = e_ref[p]
        sb = jnp.dot(r, G2_ref[p], preferred_element_type=jnp.float32)
        h1 = jax.nn.relu(
            jnp.dot(emb, pW1e, preferred_element_type=jnp.float32)
            + sb[:, 0:128] + pb1)
        pacc = pacc + jnp.dot(h1, pW2, preferred_element_type=jnp.float32)

        g1 = jax.nn.relu(
            jnp.dot(est, eW1e, preferred_element_type=jnp.float32)
            + sb[:, 128:256] + eb1)
        eacc = eacc + jnp.dot(g1, eW2, preferred_element_type=jnp.float32)

    player = pacc * (1.0 / 6.0) + pb2_ref[...]
    enemy = eacc * (1.0 / 6.0) + eb2_ref[...]

    php = jnp.mean(phl_ref[...], axis=1, keepdims=True)
    plv = jnp.mean(plvl_ref[...].astype(jnp.float32) / 100.0, axis=1,
                   keepdims=True)
    party = php * paW_ref[0:1, :] + plv * paW_ref[1:2, :] + pab_ref[...]

    g_in = jnp.concatenate([inb_ref[...], badge_ref[...], hms_ref[...]],
                           axis=1)
    gv = jax.nn.relu(jnp.dot(g_in, gW_ref[...],
                             preferred_element_type=jnp.float32)
                     + gb_ref[...])

    out = (jnp.dot(player, fWp_ref[...], preferred_element_type=jnp.float32)
           + jnp.dot(enemy, fWe_ref[...], preferred_element_type=jnp.float32)
           + jnp.dot(party, fWpa_ref[...], preferred_element_type=jnp.float32)
           + jnp.dot(map_ref[...], fWm_ref[...],
                     preferred_element_type=jnp.float32)
           + jnp.dot(gv, fWg_ref[...], preferred_element_type=jnp.float32)
           + fb_ref[...])
    out_ref[...] = jax.nn.relu(out)


def kernel(p_species, p_moves, p_ability, p_status, p_item, e_status,
           party_level, p_hp, p_lvl, p_att, p_defn, p_spe, p_spA, p_spD,
           p_pp, p_exp, e_hp, e_lvl, party_hp, inbattle, badge, hms,
           map_feat, species_emb, move_emb, ability_emb, status_emb,
           item_emb, e_status_emb, pW1, pb1, pW2, pb2, eW1, eb1, eW2, eb2,
           partyW, partyb, gW, gb, fW, fb):
    b = p_species.shape[0]
    f32 = jnp.float32

    ctab = jnp.concatenate([species_emb, move_emb, ability_emb, status_emb,
                            item_emb, e_status_emb], axis=0)
    idx_px = jnp.concatenate([
        p_species[..., None].astype(jnp.int32),
        p_moves.astype(jnp.int32) + 412,
        p_ability[..., None].astype(jnp.int32) + 767,
        p_status[..., None].astype(jnp.int32) + 845,
        p_item[..., None].astype(jnp.int32) + 851,
    ], axis=-1)
    idx_px = idx_px.transpose(1, 0, 2).reshape(-1)
    idx_est = (e_status.astype(jnp.int32) + 1228).T.reshape(-1)

    gpx, gest = _sc_gather(ctab, idx_px, idx_est)
    gpx = gpx.reshape(6, b, 128)
    gest = gest.reshape(6, b, 16)

    r_raw = jnp.concatenate(
        [p_hp, p_lvl, p_att, p_defn, p_spe, p_spA, p_spD,
         p_pp.reshape(b, 24), p_exp, e_hp, e_lvl], axis=1)

    eye6 = jnp.eye(6, dtype=f32)
    def blk(mask, w):
        return mask[:, :, None] * w[None, None, :]
    z = jnp.zeros((6, 6, 128), f32)
    Gp = jnp.concatenate([
        blk(eye6, pW1[128]), blk(eye6, pW1[129] / 100.0),
        blk(eye6, pW1[130]), blk(eye6, pW1[131]), blk(eye6, pW1[132]),
        blk(eye6, pW1[133]), blk(eye6, pW1[134]),
        blk(jnp.repeat(eye6, 4, axis=1), pW1[135] / 4.0),
        blk(eye6, pW1[136]), z, z], axis=1)
    Ge = jnp.concatenate([
        jnp.zeros((6, 72, 128), f32),
        blk(eye6, eW1[16]), blk(eye6, eW1[17] / 100.0)], axis=1)
    G2 = jnp.concatenate([Gp, Ge], axis=2)

    n = min(512, b)
    grid = (b // n,)

    def bspec(k):
        return pl.BlockSpec((n, k), lambda i: (i, 0))

    def pspec(k):
        return pl.BlockSpec((6, n, k), lambda i: (0, i, 0))

    def wspec(shape):
        nd = len(shape)
        return pl.BlockSpec(shape, lambda i: (0,) * nd)

    weights = [pW1[0:128], G2, pb1.reshape(1, 128), pW2,
               pb2.reshape(1, 128), eW1[0:16],
               eb1.reshape(1, 128), eW2, eb2.reshape(1, 128),
               partyW, partyb.reshape(1, 128), gW, gb.reshape(1, 32),
               fW[0:128], fW[128:256], fW[256:384], fW[384:439],
               fW[439:471], fb.reshape(1, 256)]

    batch_in = [(r_raw, 84), (party_level.astype(jnp.int32), 6),
                (party_hp, 6), (inbattle, 1), (badge, 8), (hms, 8),
                (map_feat, 55)]

    in_specs = ([pspec(128), pspec(16)]
                + [bspec(k) for _, k in batch_in]
                + [wspec(w.shape) for w in weights])

    return pl.pallas_call(
        _tc_body,
        grid=grid,
        in_specs=in_specs,
        out_specs=pl.BlockSpec((n, 256), lambda i: (i, 0)),
        out_shape=jax.ShapeDtypeStruct((b, 256), f32),
        compiler_params=pltpu.CompilerParams(
            dimension_semantics=("parallel",)),
    )(gpx, gest, *[a for a, _ in batch_in], *weights)


# speedup vs baseline: 3.3773x; 1.1912x over previous
import jax
import jax.numpy as jnp
from jax import lax
from jax.experimental import pallas as pl
from jax.experimental.pallas import tpu as pltpu
from jax.experimental.pallas import tpu_sc as plsc

_NC = 2
_NS = 16
_NW = _NC * _NS


def _sc_gather(ctab, idx_flat):
    v = ctab.shape[0]
    total = idx_flat.shape[0]
    per_tile = total // _NW
    n_chunks = max(1, -(-per_tile // 1536))
    while per_tile % n_chunks:
        n_chunks += 1
    chunk = per_tile // n_chunks

    mesh = plsc.VectorSubcoreMesh(
        core_axis_name="c", subcore_axis_name="s",
        num_cores=_NC, num_subcores=_NS)

    def body(ctab_hbm, idx_hbm, out_hbm, tab_v, idx_v, rows_v, osem):
        wid = lax.axis_index("s") * _NC + lax.axis_index("c")
        base = wid * per_tile
        pltpu.sync_copy(ctab_hbm, tab_v)
        iota16 = lax.iota(jnp.int32, 16)
        rots = [jnp.bitwise_and(iota16 + d, 15) for d in range(16)]
        posd = [iota16 * 16 + rots[d] for d in range(16)]

        def wb_slice(c):
            return out_hbm.at[pl.ds((base + c * chunk) * 16, chunk * 16)]

        for c in range(n_chunks):
            cur = c % 2
            if c >= 2:
                pltpu.make_async_copy(rows_v.at[cur], wb_slice(c - 2),
                                      osem).wait()
            pltpu.sync_copy(idx_hbm.at[pl.ds(base + c * chunk, chunk)],
                            idx_v)
            rv = rows_v.at[cur]

            def bg(g, carry):
                addr = idx_v[pl.ds(g * 16, 16)] * 16
                ob = g * 256
                for d in range(16):
                    vals = plsc.load_gather(tab_v, [addr + rots[d]])
                    plsc.store_scatter(rv, [posd[d] + ob], vals)
                return carry

            lax.fori_loop(0, chunk // 16, bg, 0, unroll=4)
            pltpu.async_copy(rv, wb_slice(c), osem)
        pltpu.make_async_copy(rows_v.at[(n_chunks - 1) % 2],
                              wb_slice(n_chunks - 1), osem).wait()
        if n_chunks >= 2:
            pltpu.make_async_copy(rows_v.at[n_chunks % 2],
                                  wb_slice(n_chunks - 2), osem).wait()

    f = pl.kernel(
        body,
        out_type=jax.ShapeDtypeStruct((total * 16,), jnp.float32),
        mesh=mesh,
        scratch_types=[
            pltpu.VMEM((v * 16,), jnp.float32),
            pltpu.VMEM((chunk,), jnp.int32),
            pltpu.VMEM((2, chunk * 16), jnp.float32),
            pltpu.SemaphoreType.DMA,
        ],
        compiler_params=pltpu.CompilerParams(use_tc_tiling_on_sc=False,
                                             needs_layout_passes=False),
    )
    return f(ctab.reshape(-1), idx_flat)


def _tc_body(g_ref, hp_ref, lvl_ref, att_ref, defn_ref, spe_ref, spA_ref,
             spD_ref, pp_ref, exp_ref, ehp_ref, elvl_ref, oh_ref,
             plvl_ref, phl_ref, inb_ref, badge_ref, hms_ref, map_ref,
             pW1e_ref, G2_ref, pb1_ref, pW2_ref, pb2_ref,
             eb1_ref, eW2_ref, eb2_ref,
             paW_ref, pab_ref, gW_ref, gb_ref,
             fWp_ref, fWe_ref, fWpa_ref, fWm_ref, fWg_ref, fb_ref,
             out_ref):
    n = out_ref.shape[0]
    pW1e = pW1e_ref[...]
    pb1 = pb1_ref[...]
    pW2 = pW2_ref[...]
    eb1 = eb1_ref[...]
    eW2 = eW2_ref[...]

    r = jnp.concatenate(
        [hp_ref[...], lvl_ref[...], att_ref[...], defn_ref[...],
         spe_ref[...], spA_ref[...], spD_ref[...], pp_ref[...],
         exp_ref[...], ehp_ref[...], elvl_ref[...], oh_ref[...]],
        axis=1)

    pacc = jnp.zeros((n, 128), jnp.float32)
    eacc = jnp.zeros((n, 128), jnp.float32)
    for p in range(6):
        emb = g_ref[p]
        sb = jnp.dot(r, G2_ref[p], preferred_element_type=jnp.float32)
        h1 = jax.nn.relu(
            jnp.dot(emb, pW1e, preferred_element_type=jnp.float32)
            + sb[:, 0:128] + pb1)
        pacc = pacc + jnp.dot(h1, pW2, preferred_element_type=jnp.float32)

        g1 = jax.nn.relu(sb[:, 128:256] + eb1)
        eacc = eacc + jnp.dot(g1, eW2, preferred_element_type=jnp.float32)

    player = pacc * (1.0 / 6.0) + pb2_ref[...]
    enemy = eacc * (1.0 / 6.0) + eb2_ref[...]

    php = jnp.mean(phl_ref[...], axis=1, keepdims=True)
    plv = jnp.mean(plvl_ref[...].astype(jnp.float32) / 100.0, axis=1,
                   keepdims=True)
    party = php * paW_ref[0:1, :] + plv * paW_ref[1:2, :] + pab_ref[...]

    g_in = jnp.concatenate([inb_ref[...], badge_ref[...], hms_ref[...]],
                           axis=1)
    gv = jax.nn.relu(jnp.dot(g_in, gW_ref[...],
                             preferred_element_type=jnp.float32)
                     + gb_ref[...])

    out = (jnp.dot(player, fWp_ref[...], preferred_element_type=jnp.float32)
           + jnp.dot(enemy, fWe_ref[...], preferred_element_type=jnp.float32)
           + jnp.dot(party, fWpa_ref[...], preferred_element_type=jnp.float32)
           + jnp.dot(map_ref[...], fWm_ref[...],
                     preferred_element_type=jnp.float32)
           + jnp.dot(gv, fWg_ref[...], preferred_element_type=jnp.float32)
           + fb_ref[...])
    out_ref[...] = jax.nn.relu(out)


def kernel(p_species, p_moves, p_ability, p_status, p_item, e_status,
           party_level, p_hp, p_lvl, p_att, p_defn, p_spe, p_spA, p_spD,
           p_pp, p_exp, e_hp, e_lvl, party_hp, inbattle, badge, hms,
           map_feat, species_emb, move_emb, ability_emb, status_emb,
           item_emb, e_status_emb, pW1, pb1, pW2, pb2, eW1, eb1, eW2, eb2,
           partyW, partyb, gW, gb, fW, fb):
    b = p_species.shape[0]
    f32 = jnp.float32

    ctab = jnp.concatenate([species_emb, move_emb, ability_emb, status_emb,
                            item_emb], axis=0)
    idx_px = jnp.concatenate([
        p_species[..., None].astype(jnp.int32),
        p_moves.astype(jnp.int32) + 412,
        p_ability[..., None].astype(jnp.int32) + 767,
        p_status[..., None].astype(jnp.int32) + 845,
        p_item[..., None].astype(jnp.int32) + 851,
    ], axis=-1)
    idx_px = idx_px.transpose(1, 0, 2).reshape(-1)

    gpx = _sc_gather(ctab, idx_px).reshape(6, b, 128)

    oh36 = (e_status[..., None] == jnp.arange(6)).astype(f32).reshape(b, 36)

    eye6 = jnp.eye(6, dtype=f32)

    def blk(mask, w):
        return mask[:, :, None] * w[None, None, :]

    z6 = jnp.zeros((6, 6, 128), f32)
    Gp = jnp.concatenate([
        blk(eye6, pW1[128]), blk(eye6, pW1[129] / 100.0),
        blk(eye6, pW1[130]), blk(eye6, pW1[131]), blk(eye6, pW1[132]),
        blk(eye6, pW1[133]), blk(eye6, pW1[134]),
        blk(jnp.repeat(eye6, 4, axis=1), pW1[135] / 4.0),
        blk(eye6, pW1[136]), z6, z6,
        jnp.zeros((6, 36, 128), f32)], axis=1)
    Test = jnp.dot(e_status_emb, eW1[0:16],
                   preferred_element_type=f32)
    Tblk = (eye6[:, :, None, None]
            * Test[None, None, :, :]).reshape(6, 36, 128)
    Ge = jnp.concatenate([
        jnp.zeros((6, 72, 128), f32),
        blk(eye6, eW1[16]), blk(eye6, eW1[17] / 100.0),
        Tblk], axis=1)
    G2 = jnp.concatenate([Gp, Ge], axis=2)

    n = min(512, b)
    grid = (b // n,)

    def bspec(k):
        return pl.BlockSpec((n, k), lambda i: (i, 0))

    def wspec(shape):
        nd = len(shape)
        return pl.BlockSpec(shape, lambda i: (0,) * nd)

    weights = [pW1[0:128], G2, pb1.reshape(1, 128), pW2,
               pb2.reshape(1, 128),
               eb1.reshape(1, 128), eW2, eb2.reshape(1, 128),
               partyW, partyb.reshape(1, 128), gW, gb.reshape(1, 32),
               fW[0:128], fW[128:256], fW[256:384], fW[384:439],
               fW[439:471], fb.reshape(1, 256)]

    batch_in = [(p_hp, 6), (p_lvl, 6), (p_att, 6), (p_defn, 6), (p_spe, 6),
                (p_spA, 6), (p_spD, 6), (p_pp.reshape(b, 24), 24),
                (p_exp, 6), (e_hp, 6), (e_lvl, 6), (oh36, 36),
                (party_level.astype(jnp.int32), 6), (party_hp, 6),
                (inbattle, 1), (badge, 8), (hms, 8), (map_feat, 55)]

    in_specs = ([pl.BlockSpec((6, n, 128), lambda i: (0, i, 0))]
                + [bspec(k) for _, k in batch_in]
                + [wspec(w.shape) for w in weights])

    return pl.pallas_call(
        _tc_body,
        grid=grid,
        in_specs=in_specs,
        out_specs=pl.BlockSpec((n, 256), lambda i: (i, 0)),
        out_shape=jax.ShapeDtypeStruct((b, 256), f32),
        compiler_params=pltpu.CompilerParams(
            dimension_semantics=("parallel",)),
    )(gpx, *[a for a, _ in batch_in], *weights)
